# baseline (speedup 1.0000x reference)
import jax
import jax.numpy as jnp
from jax.experimental import pallas as pl
from jax.experimental.pallas import tpu as pltpu
from jax.experimental.pallas import tpu_sc as plsc

_LANES = 16
_UNROLL = 4


def kernel(x, pos_table):
    B, S, D = x.shape
    BR = 4

    mesh = plsc.VectorSubcoreMesh(core_axis_name="c", subcore_axis_name="s")

    @pl.kernel(out_type=jax.ShapeDtypeStruct((B, S, D), x.dtype), mesh=mesh)
    def sc_add(x_hbm, p_hbm, o_hbm):
        def body(x_vmem, p_vmem, o_vmem):
            @pl.loop(0, BR)
            def _(r):
                @pl.loop(0, D, step=_UNROLL * _LANES)
                def _(c0):
                    for j in range(_UNROLL):
                        c = c0 + j * _LANES
                        pv = p_vmem[r, pl.ds(c, _LANES)]
                        for b in range(B):
                            o_vmem[b, r, pl.ds(c, _LANES)] = (
                                x_vmem[b, r, pl.ds(c, _LANES)] + pv
                            )

        pltpu.emit_pipeline(
            body,
            grid=(S // BR,),
            in_specs=[
                pl.BlockSpec((B, BR, D), index_map=lambda i: (0, i, 0)),
                pl.BlockSpec((BR, D), index_map=lambda i: (i, 0)),
            ],
            out_specs=[pl.BlockSpec((B, BR, D), index_map=lambda i: (0, i, 0))],
            core_axis_name=("c", "s"),
            dimension_semantics=(pltpu.PARALLEL,),
        )(x_hbm, p_hbm, o_hbm)

    return sc_add(x, pos_table)


# speedup vs baseline: 1.7228x; 1.7228x over previous
import jax
import jax.numpy as jnp
from jax.experimental import pallas as pl
from jax.experimental.pallas import tpu as pltpu
from jax.experimental.pallas import tpu_sc as plsc

_LANES = 16
_BS = 256
_BR = 4
_S0 = 3328


def _add_bcast(x_ref, p_ref, o_ref):
    o_ref[...] = x_ref[...] + p_ref[...]


def _tc_part(x, pos_table, nb, batch_off, seq_len):
    B, S, D = x.shape
    return pl.pallas_call(
        _add_bcast,
        grid=(seq_len // _BS,),
        in_specs=[
            pl.BlockSpec((nb, _BS, D), lambda i: (batch_off, i, 0)),
            pl.BlockSpec((_BS, D), lambda i: (i, 0)),
        ],
        out_specs=pl.BlockSpec((nb, _BS, D), lambda i: (0, i, 0)),
        out_shape=jax.ShapeDtypeStruct((nb, seq_len, D), x.dtype),
        compiler_params=pltpu.CompilerParams(
            dimension_semantics=("parallel",),
        ),
    )(x, pos_table)


def _sc_part(x, pos_table):
    B, S, D = x.shape
    n_rows = S - _S0
    blk0 = _S0 // _BR

    mesh = plsc.VectorSubcoreMesh(core_axis_name="c", subcore_axis_name="s")

    @pl.kernel(
        out_type=jax.ShapeDtypeStruct((1, n_rows, D), x.dtype), mesh=mesh
    )
    def sc_add(x_hbm, p_hbm, o_hbm):
        def body(x_vmem, p_vmem, o_vmem):
            @pl.loop(0, _BR)
            def _(r):
                @pl.loop(0, D, step=4 * _LANES)
                def _(c0):
                    for j in range(4):
                        c = c0 + j * _LANES
                        o_vmem[0, r, pl.ds(c, _LANES)] = (
                            x_vmem[0, r, pl.ds(c, _LANES)]
                            + p_vmem[r, pl.ds(c, _LANES)]
                        )

        pltpu.emit_pipeline(
            body,
            grid=(n_rows // _BR,),
            in_specs=[
                pl.BlockSpec((1, _BR, D), index_map=lambda i: (3, blk0 + i, 0)),
                pl.BlockSpec((_BR, D), index_map=lambda i: (blk0 + i, 0)),
            ],
            out_specs=[pl.BlockSpec((1, _BR, D), index_map=lambda i: (0, i, 0))],
            core_axis_name=("c", "s"),
            dimension_semantics=(pltpu.PARALLEL,),
        )(x_hbm, p_hbm, o_hbm)

    return sc_add(x, pos_table)


def kernel(x, pos_table):
    B, S, D = x.shape
    tc1 = _tc_part(x, pos_table, 3, 0, S)
    tc2 = _tc_part(x, pos_table, 1, 3, _S0)
    sc = _sc_part(x, pos_table)
    flat = jnp.concatenate(
        [tc1.reshape(-1, D), tc2.reshape(-1, D), sc.reshape(-1, D)], axis=0
    )
    return flat.reshape(B, S, D)


# speedup vs baseline: 3.3960x; 1.9712x over previous
import jax
import jax.numpy as jnp
from jax.experimental import pallas as pl
from jax.experimental.pallas import tpu as pltpu


def _pe_add_kernel(x_ref, p_ref, o_ref):
    o_ref[...] = x_ref[...] + p_ref[...]


def kernel(x, pos_table):
    B, S, D = x.shape
    BS = 512
    grid = (S // BS, B)
    return pl.pallas_call(
        _pe_add_kernel,
        grid=grid,
        in_specs=[
            pl.BlockSpec((1, BS, D), lambda i, b: (b, i, 0)),
            pl.BlockSpec((BS, D), lambda i, b: (i, 0)),
        ],
        out_specs=pl.BlockSpec((1, BS, D), lambda i, b: (b, i, 0)),
        out_shape=jax.ShapeDtypeStruct((B, S, D), x.dtype),
        compiler_params=pltpu.CompilerParams(
            dimension_semantics=("arbitrary", "arbitrary"),
        ),
    )(x, pos_table)


# speedup vs baseline: 3.6936x; 1.0876x over previous
import jax
import jax.numpy as jnp
from jax.experimental import pallas as pl
from jax.experimental.pallas import tpu as pltpu


def _pe_add_kernel(x_ref, p_ref, o_ref):
    o_ref[...] = x_ref[...] + p_ref[...]


def kernel(x, pos_table):
    B, S, D = x.shape
    BS = 128
    grid = (S // BS,)
    return pl.pallas_call(
        _pe_add_kernel,
        grid=grid,
        in_specs=[
            pl.BlockSpec((B, BS, D), lambda i: (0, i, 0)),
            pl.BlockSpec((BS, D), lambda i: (i, 0)),
        ],
        out_specs=pl.BlockSpec((B, BS, D), lambda i: (0, i, 0)),
        out_shape=jax.ShapeDtypeStruct((B, S, D), x.dtype),
        compiler_params=pltpu.CompilerParams(
            dimension_semantics=("parallel",),
        ),
    )(x, pos_table)


# speedup vs baseline: 3.9443x; 1.0679x over previous
import jax
import jax.numpy as jnp
from jax.experimental import pallas as pl
from jax.experimental.pallas import tpu as pltpu


def _pe_add_kernel(x_ref, p_ref, o_ref):
    o_ref[...] = x_ref[...] + p_ref[...]


def kernel(x, pos_table):
    B, S, D = x.shape
    BS = 512
    grid = (S // BS,)
    return pl.pallas_call(
        _pe_add_kernel,
        grid=grid,
        in_specs=[
            pl.BlockSpec((B, BS, D), lambda i: (0, i, 0)),
            pl.BlockSpec((BS, D), lambda i: (i, 0)),
        ],
        out_specs=pl.BlockSpec((B, BS, D), lambda i: (0, i, 0)),
        out_shape=jax.ShapeDtypeStruct((B, S, D), x.dtype),
        compiler_params=pltpu.CompilerParams(
            dimension_semantics=("parallel",),
        ),
    )(x, pos_table)


# speedup vs baseline: 3.9455x; 1.0003x over previous
import jax
import jax.numpy as jnp
from jax.experimental import pallas as pl
from jax.experimental.pallas import tpu as pltpu


def _pe_add_kernel(x_ref, p_ref, o_ref):
    o_ref[...] = x_ref[...] + p_ref[...]


def kernel(x, pos_table):
    B, S, D = x.shape
    BS = 512
    grid = (S // BS,)
    return pl.pallas_call(
        _pe_add_kernel,
        grid=grid,
        in_specs=[
            pl.BlockSpec((B, BS, D), lambda i: (0, i, 0)),
            pl.BlockSpec((BS, D), lambda i: (i, 0)),
        ],
        out_specs=pl.BlockSpec((B, BS, D), lambda i: (0, i, 0)),
        out_shape=jax.ShapeDtypeStruct((B, S, D), x.dtype),
        compiler_params=pltpu.CompilerParams(
            dimension_semantics=("arbitrary",),
        ),
    )(x, pos_table)
